# baseline (speedup 1.0000x reference)
import functools

import jax
import jax.numpy as jnp
from jax.experimental import pallas as pl
from jax.experimental.pallas import tpu as pltpu


def _gate_kernel(posT_ref, w1t_ref, b1_ref, w2t_ref, b2_ref, gate_ref):
    h = jnp.dot(w1t_ref[...], posT_ref[...],
                preferred_element_type=jnp.float32) + b1_ref[...]
    h = jnp.maximum(h, 0.0)
    g = jnp.dot(w2t_ref[...].astype(jnp.bfloat16), h.astype(jnp.bfloat16),
                preferred_element_type=jnp.float32) + b2_ref[...]
    gate_ref[...] = jnp.maximum(g, 0.0)


def _main_kernel(x_ref, gate_ref, wmat_ref, bconv_ref, wcgt_ref, bcg_ref,
                 out_ref, *, H, W, K):
    Cin = x_ref.shape[1]
    HW = x_ref.shape[2]
    p = K // 2
    maxoff = p * W + p

    x = x_ref[0]
    col = jax.lax.broadcasted_iota(jnp.int32, (1, HW), 1) % W
    zero = jnp.zeros((), x.dtype)
    zpad = jnp.zeros((Cin, maxoff), x.dtype)

    groups = []
    for kj in range(K):
        dj = kj - p
        if dj < 0:
            xm = jnp.where(col < W + dj, x, zero)
        elif dj > 0:
            xm = jnp.where(col >= dj, x, zero)
        else:
            xm = x
        xp = jnp.concatenate([zpad, xm, zpad], axis=1)
        for ki in range(K):
            off = (ki - p) * W + dj
            groups.append(xp[:, maxoff + off: maxoff + off + HW])
    patches = jnp.concatenate(groups, axis=0)

    y = jnp.dot(wmat_ref[...], patches,
                preferred_element_type=jnp.float32) + bconv_ref[...]

    gated = (y * gate_ref[...]).astype(jnp.bfloat16)

    out = jnp.dot(wcgt_ref[...], gated,
                  preferred_element_type=jnp.float32) + bcg_ref[...]
    out_ref[0] = out


def kernel(x_nchw, wconv, bconv, pos, w1, b1, w2, b2, wcg, bcg):
    N, Cin, H, W = x_nchw.shape
    K = wconv.shape[0]
    Cg = wconv.shape[3]
    Cout = wcg.shape[1]
    HW = H * W

    wmat = jnp.transpose(wconv, (3, 1, 0, 2)).reshape(
        Cg, K * K * Cin).astype(jnp.bfloat16)
    bconv_c = bconv.reshape(Cg, 1)
    posT = pos.T
    w1t = w1.T
    w2t = w2.T
    b1c = b1.reshape(Cg, 1)
    b2c = b2.reshape(Cg, 1)
    wcgT = wcg.T.astype(jnp.bfloat16)
    bcg_c = bcg.reshape(Cout, 1)

    x_flat = x_nchw.reshape(N, Cin, HW).astype(jnp.bfloat16)

    vmem = pl.BlockSpec(memory_space=pltpu.MemorySpace.VMEM)

    gate = pl.pallas_call(
        _gate_kernel,
        out_shape=jax.ShapeDtypeStruct((Cg, HW), jnp.float32),
        in_specs=[vmem] * 5,
        out_specs=vmem,
    )(posT, w1t, b1c, w2t, b2c)

    body = functools.partial(_main_kernel, H=H, W=W, K=K)
    flops = 2 * N * HW * (K * K * Cin * Cg + Cg * Cout) + N * Cg * HW
    bytes_accessed = 2 * N * Cin * HW + 4 * (N * Cout * HW + Cg * HW
                     + Cg + Cout) + 2 * (Cg * K * K * Cin + Cout * Cg)

    out_flat = pl.pallas_call(
        body,
        out_shape=jax.ShapeDtypeStruct((N, Cout, HW), jnp.float32),
        grid=(N,),
        in_specs=[
            pl.BlockSpec((1, Cin, HW), lambda n: (n, 0, 0)),
            pl.BlockSpec((Cg, HW), lambda n: (0, 0)),
            pl.BlockSpec((Cg, K * K * Cin), lambda n: (0, 0)),
            pl.BlockSpec((Cg, 1), lambda n: (0, 0)),
            pl.BlockSpec((Cout, Cg), lambda n: (0, 0)),
            pl.BlockSpec((Cout, 1), lambda n: (0, 0)),
        ],
        out_specs=pl.BlockSpec((1, Cout, HW), lambda n: (n, 0, 0)),
        compiler_params=pltpu.CompilerParams(
            dimension_semantics=("parallel",),
            vmem_limit_bytes=64 * 1024 * 1024),
        cost_estimate=pl.CostEstimate(flops=flops, transcendentals=0,
                                      bytes_accessed=bytes_accessed),
    )(x_flat, gate, wmat, bconv_c, wcgT, bcg_c)

    return out_flat.reshape(N, Cout, H, W)


# speedup vs baseline: 1.7879x; 1.7879x over previous
import functools

import jax
import jax.numpy as jnp
from jax.experimental import pallas as pl
from jax.experimental.pallas import tpu as pltpu


def _gate_kernel(posT_ref, w1t_ref, b1_ref, w2t_ref, b2_ref, gate_ref):
    h = jnp.dot(w1t_ref[...], posT_ref[...],
                preferred_element_type=jnp.float32) + b1_ref[...]
    h = jnp.maximum(h, 0.0)
    g = jnp.dot(w2t_ref[...].astype(jnp.bfloat16), h.astype(jnp.bfloat16),
                preferred_element_type=jnp.float32) + b2_ref[...]
    gate_ref[...] = jnp.maximum(g, 0.0)


def _main_kernel(x_ref, gate_ref, wmat_ref, bconv_ref, wcgt_ref, bcg_ref,
                 out_ref, *, H, W, K):
    Cin = x_ref.shape[1]
    HW = H * W
    p = K // 2
    maxoff = p * W + p

    x = x_ref[0].reshape(Cin, HW).astype(jnp.bfloat16)
    col = jax.lax.broadcasted_iota(jnp.int32, (1, HW), 1) % W
    zero = jnp.zeros((), x.dtype)
    zpad = jnp.zeros((Cin, maxoff), x.dtype)

    groups = []
    for kj in range(K):
        dj = kj - p
        if dj < 0:
            xm = jnp.where(col < W + dj, x, zero)
        elif dj > 0:
            xm = jnp.where(col >= dj, x, zero)
        else:
            xm = x
        xp = jnp.concatenate([zpad, xm, zpad], axis=1)
        for ki in range(K):
            off = (ki - p) * W + dj
            groups.append(xp[:, maxoff + off: maxoff + off + HW])
    patches = jnp.concatenate(groups, axis=0)

    y = jnp.dot(wmat_ref[...], patches,
                preferred_element_type=jnp.float32) + bconv_ref[...]

    gated = (y * gate_ref[...]).astype(jnp.bfloat16)

    out = jnp.dot(wcgt_ref[...], gated,
                  preferred_element_type=jnp.float32) + bcg_ref[...]
    out_ref[0] = out.reshape(out_ref.shape[1], H, W)


def kernel(x_nchw, wconv, bconv, pos, w1, b1, w2, b2, wcg, bcg):
    N, Cin, H, W = x_nchw.shape
    K = wconv.shape[0]
    Cg = wconv.shape[3]
    Cout = wcg.shape[1]
    HW = H * W

    wmat = jnp.transpose(wconv, (3, 1, 0, 2)).reshape(
        Cg, K * K * Cin).astype(jnp.bfloat16)
    bconv_c = bconv.reshape(Cg, 1)
    posT = pos.T
    w1t = w1.T
    w2t = w2.T
    b1c = b1.reshape(Cg, 1)
    b2c = b2.reshape(Cg, 1)
    wcgT = wcg.T.astype(jnp.bfloat16)
    bcg_c = bcg.reshape(Cout, 1)

    vmem = pl.BlockSpec(memory_space=pltpu.MemorySpace.VMEM)

    gate = pl.pallas_call(
        _gate_kernel,
        out_shape=jax.ShapeDtypeStruct((Cg, HW), jnp.float32),
        in_specs=[vmem] * 5,
        out_specs=vmem,
    )(posT, w1t, b1c, w2t, b2c)

    body = functools.partial(_main_kernel, H=H, W=W, K=K)
    flops = 2 * N * HW * (K * K * Cin * Cg + Cg * Cout) + N * Cg * HW
    bytes_accessed = 4 * (N * Cin * HW + N * Cout * HW + Cg * HW
                          + Cg + Cout) + 2 * (Cg * K * K * Cin + Cout * Cg)

    half = N // 2
    out = pl.pallas_call(
        body,
        out_shape=jax.ShapeDtypeStruct((N, Cout, H, W), jnp.float32),
        grid=(2, half),
        in_specs=[
            pl.BlockSpec((1, Cin, H, W), lambda c, n: (c * half + n, 0, 0, 0)),
            pl.BlockSpec((Cg, HW), lambda c, n: (0, 0)),
            pl.BlockSpec((Cg, K * K * Cin), lambda c, n: (0, 0)),
            pl.BlockSpec((Cg, 1), lambda c, n: (0, 0)),
            pl.BlockSpec((Cout, Cg), lambda c, n: (0, 0)),
            pl.BlockSpec((Cout, 1), lambda c, n: (0, 0)),
        ],
        out_specs=pl.BlockSpec((1, Cout, H, W),
                               lambda c, n: (c * half + n, 0, 0, 0)),
        compiler_params=pltpu.CompilerParams(
            dimension_semantics=("parallel", "arbitrary"),
            vmem_limit_bytes=64 * 1024 * 1024),
        cost_estimate=pl.CostEstimate(flops=flops, transcendentals=0,
                                      bytes_accessed=bytes_accessed),
    )(x_nchw, gate, wmat, bconv_c, wcgT, bcg_c)

    return out


# speedup vs baseline: 1.8636x; 1.0423x over previous
import functools

import jax
import jax.numpy as jnp
from jax.experimental import pallas as pl
from jax.experimental.pallas import tpu as pltpu


def _gate_kernel(posT_ref, w1t_ref, b1_ref, w2t_ref, b2_ref, gate_ref):
    h = jnp.dot(w1t_ref[...], posT_ref[...],
                preferred_element_type=jnp.float32) + b1_ref[...]
    h = jnp.maximum(h, 0.0)
    g = jnp.dot(w2t_ref[...].astype(jnp.bfloat16), h.astype(jnp.bfloat16),
                preferred_element_type=jnp.float32) + b2_ref[...]
    gate_ref[...] = jnp.maximum(g, 0.0)


def _main_kernel(x_ref, gate_ref, wmat_ref, bconv_ref, wcgt_ref, bcg_ref,
                 out_ref, *, H, W, K):
    Cin = x_ref.shape[1]
    HW = H * W
    p = K // 2
    maxoff = p * W + p

    x = x_ref[0].astype(jnp.bfloat16).reshape(Cin, HW)
    col = jax.lax.broadcasted_iota(jnp.int32, (1, HW), 1) % W
    zero = jnp.zeros((), x.dtype)
    zpad = jnp.zeros((Cin, maxoff), x.dtype)

    groups = []
    for kj in range(K):
        dj = kj - p
        if dj < 0:
            xm = jnp.where(col < W + dj, x, zero)
        elif dj > 0:
            xm = jnp.where(col >= dj, x, zero)
        else:
            xm = x
        xp = jnp.concatenate([zpad, xm, zpad], axis=1)
        for ki in range(K):
            off = (ki - p) * W + dj
            groups.append(xp[:, maxoff + off: maxoff + off + HW])
    patches = jnp.concatenate(groups, axis=0)

    y = jnp.dot(wmat_ref[...], patches,
                preferred_element_type=jnp.float32) + bconv_ref[...]

    gated = (y * gate_ref[...]).astype(jnp.bfloat16)

    out = jnp.dot(wcgt_ref[...], gated,
                  preferred_element_type=jnp.float32) + bcg_ref[...]
    out_ref[0] = out.reshape(out_ref.shape[1], H, W)


def kernel(x_nchw, wconv, bconv, pos, w1, b1, w2, b2, wcg, bcg):
    N, Cin, H, W = x_nchw.shape
    K = wconv.shape[0]
    Cg = wconv.shape[3]
    Cout = wcg.shape[1]
    HW = H * W

    wmat = jnp.transpose(wconv, (3, 1, 0, 2)).reshape(
        Cg, K * K * Cin).astype(jnp.bfloat16)
    bconv_c = bconv.reshape(Cg, 1)
    posT = pos.T
    w1t = w1.T
    w2t = w2.T
    b1c = b1.reshape(Cg, 1)
    b2c = b2.reshape(Cg, 1)
    wcgT = wcg.T.astype(jnp.bfloat16)
    bcg_c = bcg.reshape(Cout, 1)

    vmem = pl.BlockSpec(memory_space=pltpu.MemorySpace.VMEM)

    gate = pl.pallas_call(
        _gate_kernel,
        out_shape=jax.ShapeDtypeStruct((Cg, HW), jnp.float32),
        in_specs=[vmem] * 5,
        out_specs=vmem,
    )(posT, w1t, b1c, w2t, b2c)

    body = functools.partial(_main_kernel, H=H, W=W, K=K)
    flops = 2 * N * HW * (K * K * Cin * Cg + Cg * Cout) + N * Cg * HW
    bytes_accessed = 4 * (N * Cin * HW + N * Cout * HW + Cg * HW
                          + Cg + Cout) + 2 * (Cg * K * K * Cin + Cout * Cg)

    half = N // 2
    out = pl.pallas_call(
        body,
        out_shape=jax.ShapeDtypeStruct((N, Cout, H, W), jnp.float32),
        grid=(2, half),
        in_specs=[
            pl.BlockSpec((1, Cin, H, W), lambda c, n: (c * half + n, 0, 0, 0)),
            pl.BlockSpec((Cg, HW), lambda c, n: (0, 0)),
            pl.BlockSpec((Cg, K * K * Cin), lambda c, n: (0, 0)),
            pl.BlockSpec((Cg, 1), lambda c, n: (0, 0)),
            pl.BlockSpec((Cout, Cg), lambda c, n: (0, 0)),
            pl.BlockSpec((Cout, 1), lambda c, n: (0, 0)),
        ],
        out_specs=pl.BlockSpec((1, Cout, H, W),
                               lambda c, n: (c * half + n, 0, 0, 0)),
        compiler_params=pltpu.CompilerParams(
            dimension_semantics=("parallel", "arbitrary"),
            vmem_limit_bytes=64 * 1024 * 1024),
        cost_estimate=pl.CostEstimate(flops=flops, transcendentals=0,
                                      bytes_accessed=bytes_accessed),
    )(x_nchw, gate, wmat, bconv_c, wcgT, bcg_c)

    return out


# speedup vs baseline: 2.5535x; 1.3701x over previous
import functools

import jax
import jax.numpy as jnp
from jax.experimental import pallas as pl
from jax.experimental.pallas import tpu as pltpu


def _main_kernel(x_ref, pos_ref, w1_ref, b1_ref, w2_ref, b2_ref,
                 wmat_ref, bconv_ref, wcg_ref, bcg_ref,
                 out_ref, gate_s, wcgt_s, bconv_s, bcg_s, *, H, W, K):
    Cin = x_ref.shape[1]
    HW = H * W
    p = K // 2
    maxoff = p * W + p
    n = pl.program_id(1)

    @pl.when(n == 0)
    def _prep():
        posT = pos_ref[...].T
        w1t = w1_ref[...].T
        b1c = b1_ref[...].T
        h = jnp.dot(w1t, posT, preferred_element_type=jnp.float32) + b1c
        h = jnp.maximum(h, 0.0)
        g = jnp.dot(w2_ref[...].T.astype(jnp.bfloat16), h.astype(jnp.bfloat16),
                    preferred_element_type=jnp.float32) + b2_ref[...].T
        gate_s[...] = jnp.maximum(g, 0.0)
        wcgt_s[...] = wcg_ref[...].T.astype(jnp.bfloat16)
        bconv_s[...] = bconv_ref[...].T
        bcg_s[...] = bcg_ref[...].T

    x = x_ref[0].astype(jnp.bfloat16).reshape(Cin, HW)
    col = jax.lax.broadcasted_iota(jnp.int32, (1, HW), 1) % W
    zpad = jnp.zeros((Cin, maxoff), x.dtype)
    xpad = jnp.concatenate([zpad, x, zpad], axis=1)

    groups = []
    for kj in range(K):
        dj = kj - p
        for ki in range(K):
            off = (ki - p) * W + dj
            patch = xpad[:, maxoff + off: maxoff + off + HW]
            if dj < 0:
                patch = jnp.where(col >= -dj, patch, jnp.zeros((), patch.dtype))
            elif dj > 0:
                patch = jnp.where(col < W - dj, patch, jnp.zeros((), patch.dtype))
            groups.append(patch)
    patches = jnp.concatenate(groups, axis=0)

    y = jnp.dot(wmat_ref[...], patches,
                preferred_element_type=jnp.float32) + bconv_s[...]

    gated = (y * gate_s[...]).astype(jnp.bfloat16)

    out = jnp.dot(wcgt_s[...], gated,
                  preferred_element_type=jnp.float32) + bcg_s[...]
    out_ref[0] = out.reshape(out_ref.shape[1], H, W)


def kernel(x_nchw, wconv, bconv, pos, w1, b1, w2, b2, wcg, bcg):
    N, Cin, H, W = x_nchw.shape
    K = wconv.shape[0]
    Cg = wconv.shape[3]
    Cout = wcg.shape[1]
    HW = H * W

    wmat = jnp.transpose(wconv, (3, 1, 0, 2)).reshape(
        Cg, K * K * Cin).astype(jnp.bfloat16)
    bconv_r = bconv.reshape(1, Cg)
    b1r = b1.reshape(1, Cg)
    b2r = b2.reshape(1, Cg)
    bcg_r = bcg.reshape(1, Cout)

    body = functools.partial(_main_kernel, H=H, W=W, K=K)
    flops = 2 * N * HW * (K * K * Cin * Cg + Cg * Cout) + N * Cg * HW
    bytes_accessed = 4 * (N * Cin * HW + N * Cout * HW
                          + Cg + Cout) + 2 * (Cg * K * K * Cin + Cout * Cg)

    half = N // 2
    cfix = lambda c, n: (0, 0)
    out = pl.pallas_call(
        body,
        out_shape=jax.ShapeDtypeStruct((N, Cout, H, W), jnp.float32),
        grid=(2, half),
        in_specs=[
            pl.BlockSpec((1, Cin, H, W), lambda c, n: (c * half + n, 0, 0, 0)),
            pl.BlockSpec((HW, 2), cfix),
            pl.BlockSpec((2, Cg), cfix),
            pl.BlockSpec((1, Cg), cfix),
            pl.BlockSpec((Cg, Cg), cfix),
            pl.BlockSpec((1, Cg), cfix),
            pl.BlockSpec((Cg, K * K * Cin), cfix),
            pl.BlockSpec((1, Cg), cfix),
            pl.BlockSpec((Cg, Cout), cfix),
            pl.BlockSpec((1, Cout), cfix),
        ],
        out_specs=pl.BlockSpec((1, Cout, H, W),
                               lambda c, n: (c * half + n, 0, 0, 0)),
        scratch_shapes=[
            pltpu.VMEM((Cg, HW), jnp.float32),
            pltpu.VMEM((Cout, Cg), jnp.bfloat16),
            pltpu.VMEM((Cg, 1), jnp.float32),
            pltpu.VMEM((Cout, 1), jnp.float32),
        ],
        compiler_params=pltpu.CompilerParams(
            dimension_semantics=("parallel", "arbitrary"),
            vmem_limit_bytes=64 * 1024 * 1024),
        cost_estimate=pl.CostEstimate(flops=flops, transcendentals=0,
                                      bytes_accessed=bytes_accessed),
    )(x_nchw, pos, w1, b1r, w2, b2r, wmat, bconv_r, wcg, bcg_r)

    return out
